# Initial kernel scaffold; baseline (speedup 1.0000x reference)
import jax
import jax.numpy as jnp
from jax.experimental import pallas as pl


def kernel(entity_emb, relation_emb, qgnn_weight, bn_gamma, bn_beta, bn1_gamma, bn1_beta, adj_values, edge_index, e1_idx, r_idx, lst_ents):
    raise NotImplementedError("write your pallas kernel here")


# speedup vs baseline: 3.5395x; 3.5395x over previous
import functools

import jax
import jax.numpy as jnp
from jax import lax
from jax.experimental import pallas as pl
from jax.experimental.pallas import tpu as pltpu
from jax.experimental.pallas import tpu_sc as plsc

N_ENT = 10000
EMB = 128
N_EDGES = 320000
BATCH = 4096

NC, NS, L = 2, 16, 16
NW = NC * NS
G = 128
NG = N_EDGES // G
ACC_ROWS = 10240
ROWS_PER_TILE = ACC_ROWS // NS
E1_PER_TILE = BATCH // NS
R_PER_W = BATCH // NW


def _tc_support_body(x_ref, w_ref, o_ref):
    w = w_ref[...]
    r = w[:, 0:32]
    i = w[:, 32:64]
    j = w[:, 64:96]
    k = w[:, 96:128]
    r2 = jnp.concatenate([r, -i, -j, -k], axis=0)
    i2 = jnp.concatenate([i, r, -k, j], axis=0)
    j2 = jnp.concatenate([j, k, r, -i], axis=0)
    k2 = jnp.concatenate([k, -j, i, r], axis=0)
    ham = jnp.concatenate([r2, i2, j2, k2], axis=1)
    o_ref[...] = jnp.dot(x_ref[...], ham, preferred_element_type=jnp.float32)


def _tc_support(x, w):
    blk = 1000
    return pl.pallas_call(
        _tc_support_body,
        grid=(N_ENT // blk,),
        in_specs=[
            pl.BlockSpec((blk, EMB), lambda i: (i, 0)),
            pl.BlockSpec((EMB // 4, EMB), lambda i: (0, 0)),
        ],
        out_specs=pl.BlockSpec((blk, EMB), lambda i: (i, 0)),
        out_shape=jax.ShapeDtypeStruct((N_ENT, EMB), jnp.float32),
    )(x, w)


_MESH = plsc.VectorSubcoreMesh(core_axis_name="c", subcore_axis_name="s")


@functools.partial(
    pl.kernel,
    out_type=[
        jax.ShapeDtypeStruct((NC, ACC_ROWS, EMB), jnp.float32),
        jax.ShapeDtypeStruct((NC, BATCH, EMB), jnp.float32),
        jax.ShapeDtypeStruct((BATCH, EMB), jnp.float32),
    ],
    mesh=_MESH,
    scratch_types=[
        pltpu.VMEM((G,), jnp.int32),
        pltpu.VMEM((G,), jnp.int32),
        pltpu.VMEM((G,), jnp.float32),
        pltpu.VMEM((G, EMB), jnp.float32),
        pltpu.VMEM_SHARED((ACC_ROWS, EMB), jnp.float32),
        pltpu.SemaphoreType.DMA,
    ],
)
def _sc_spmm(support, src, dst, adj, e1, ridx, rel,
             acc_out, he_out, rg_out,
             src_v, dst_v, adj_v, rows_v, acc_sh, sem):
    cid = lax.axis_index("c")
    sid = lax.axis_index("s")
    wid = sid * NC + cid

    zero = jnp.zeros((L,), jnp.float32)

    def _zbody(t, _):
        r = t // (EMB // L)
        c = (t % (EMB // L)) * L
        rows_v[r, pl.ds(c, L)] = zero
        return 0
    lax.fori_loop(0, G * (EMB // L), _zbody, 0)
    for kk in range(ROWS_PER_TILE // G):
        pltpu.sync_copy(rows_v, acc_sh.at[pl.ds(sid * ROWS_PER_TILE + kk * G, G)])
    plsc.subcore_barrier()

    nloops = (NG - wid + NW - 1) // NW

    def _group(t, _):
        base = (wid + t * NW) * G
        pltpu.sync_copy(src.at[pl.ds(base, G)], src_v)
        pltpu.sync_copy(dst.at[pl.ds(base, G)], dst_v)
        pltpu.sync_copy(adj.at[pl.ds(base, G)], adj_v)
        pltpu.async_copy(support.at[src_v], rows_v, sem).wait()

        def _scale(b, _):
            a16 = adj_v[pl.ds(b * L, L)]
            for t in range(L):
                e = b * L + t
                a = a16[t]
                for jj in range(EMB // L):
                    rows_v[e, pl.ds(jj * L, L)] = rows_v[e, pl.ds(jj * L, L)] * a
            return 0
        lax.fori_loop(0, G // L, _scale, 0)

        pltpu.sync_copy(rows_v, acc_sh.at[dst_v], add=True)
        return 0
    lax.fori_loop(0, nloops, _group, 0)
    plsc.subcore_barrier()

    for kk in range(ROWS_PER_TILE // G):
        off = sid * ROWS_PER_TILE + kk * G
        pltpu.sync_copy(acc_sh.at[pl.ds(off, G)], rows_v)
        pltpu.sync_copy(rows_v, acc_out.at[cid, pl.ds(off, G)])

    for q in range(E1_PER_TILE // G):
        off = sid * E1_PER_TILE + q * G
        pltpu.sync_copy(e1.at[pl.ds(off, G)], src_v)
        pltpu.async_copy(acc_sh.at[src_v], rows_v, sem).wait()
        pltpu.sync_copy(rows_v, he_out.at[cid, pl.ds(off, G)])

    roff = wid * R_PER_W
    pltpu.sync_copy(ridx.at[pl.ds(roff, R_PER_W)], src_v)
    pltpu.async_copy(rel.at[src_v], rows_v, sem).wait()
    pltpu.sync_copy(rows_v, rg_out.at[pl.ds(roff, R_PER_W)])


def _tc_bn_body(a0_ref, a1_ref, he0_ref, he1_ref, rg_ref,
                g_ref, b_ref, g1_ref, b1_ref, x1_ref, hr_ref):
    s = a0_ref[...] + a1_ref[...]
    m = jnp.sum(s, axis=0, keepdims=True) / N_ENT
    d = s - m
    valid = lax.broadcasted_iota(jnp.int32, (ACC_ROWS, 1), 0) < N_ENT
    v = jnp.sum(jnp.where(valid, d * d, 0.0), axis=0, keepdims=True) / N_ENT
    scale = g_ref[...] / jnp.sqrt(v + 1e-5)
    x1_ref[...] = jnp.tanh(d[0:N_ENT, :] * scale + b_ref[...])

    h_raw = he0_ref[...] + he1_ref[...]
    h = jnp.tanh((h_raw - m) * scale + b_ref[...])
    hr0 = h * rg_ref[...]
    m1 = jnp.sum(hr0, axis=0, keepdims=True) / BATCH
    d1 = hr0 - m1
    v1 = jnp.sum(d1 * d1, axis=0, keepdims=True) / BATCH
    hr_ref[...] = d1 * (g1_ref[...] / jnp.sqrt(v1 + 1e-5)) + b1_ref[...]


def _tc_bn(a0, a1, he0, he1, rg, g, b, g1, b1):
    return pl.pallas_call(
        _tc_bn_body,
        out_shape=[
            jax.ShapeDtypeStruct((N_ENT, EMB), jnp.float32),
            jax.ShapeDtypeStruct((BATCH, EMB), jnp.float32),
        ],
    )(a0, a1, he0, he1, rg, g, b, g1, b1)


def _tc_score_body(hr_ref, x1_ref, o_ref):
    o_ref[...] = jax.nn.sigmoid(
        lax.dot_general(hr_ref[...], x1_ref[...],
                        dimension_numbers=(((1,), (1,)), ((), ())),
                        preferred_element_type=jnp.float32))


def _tc_score(hr, x1):
    blk = 512
    return pl.pallas_call(
        _tc_score_body,
        grid=(BATCH // blk,),
        in_specs=[
            pl.BlockSpec((blk, EMB), lambda i: (i, 0)),
            pl.BlockSpec((N_ENT, EMB), lambda i: (0, 0)),
        ],
        out_specs=pl.BlockSpec((blk, N_ENT), lambda i: (i, 0)),
        out_shape=jax.ShapeDtypeStruct((BATCH, N_ENT), jnp.float32),
    )(hr, x1)


def kernel(entity_emb, relation_emb, qgnn_weight, bn_gamma, bn_beta,
           bn1_gamma, bn1_beta, adj_values, edge_index, e1_idx, r_idx,
           lst_ents):
    support = _tc_support(entity_emb, qgnn_weight)

    src = edge_index[1].astype(jnp.int32)
    dst = edge_index[0].astype(jnp.int32)
    acc, he, rg = _sc_spmm(support, src, dst, adj_values,
                           e1_idx.astype(jnp.int32), r_idx.astype(jnp.int32),
                           relation_emb)

    x1, hr = _tc_bn(acc[0], acc[1], he[0], he[1], rg,
                    bn_gamma.reshape(1, EMB), bn_beta.reshape(1, EMB),
                    bn1_gamma.reshape(1, EMB), bn1_beta.reshape(1, EMB))
    return _tc_score(hr, x1)
